# baseline (speedup 1.0000x reference)
import functools

import jax
import jax.numpy as jnp
from jax import lax
from jax.experimental import pallas as pl
from jax.experimental.pallas import tpu as pltpu
from jax.experimental.pallas import tpu_sc as plsc

N = 100000
D = 128
NSEG = 256
SROWS = 160
NSUPER = N // SROWS
NBUF = 4
UNROLL = 16
NC = 2
NS = 16
NW = NC * NS


def _seg_body(x_hbm, batch_hbm, sums_out, cnts_out,
              xbuf, idxbuf, accloc, cntloc, rowids, stage, acc, sem_in):
    cid = lax.axis_index("c")
    sid = lax.axis_index("s")
    wid = sid * NC + cid

    z16 = jnp.zeros((16,), jnp.float32)
    iota = lax.iota(jnp.int32, 16)
    for r in range(16):
        for j in range(D // 16):
            stage[r, pl.ds(j * 16, 16)] = z16
    @pl.loop(0, NSEG)
    def _zacc(r):
        for j in range(D // 16):
            accloc[r, pl.ds(j * 16, 16)] = z16
    for j in range(NSEG // 16):
        cntloc[pl.ds(j * 16, 16)] = z16
    for h in range(2):
        for j in range(8):
            rowids[h, pl.ds(j * 16, 16)] = iota + (h * 128 + j * 16)
    pltpu.sync_copy(stage, acc.at[pl.ds(sid * 16, 16)])

    plsc.subcore_barrier()

    base = NSUPER // NW
    rem = NSUPER - base * NW
    ntrip = base + jnp.where(wid < rem, 1, 0)
    o16 = jnp.ones((16,), jnp.float32)
    cols = [iota + (j * 16) for j in range(D // 16)]

    def issue_in(k, b):
        s = wid + k * NW
        pltpu.async_copy(x_hbm.at[pl.ds(s * SROWS, SROWS)], xbuf.at[b],
                         sem_in.at[b])
        for h in range(2):
            pltpu.async_copy(batch_hbm.at[pl.ds(s * SROWS + h * 80, 80)],
                             idxbuf.at[b].at[h], sem_in.at[b])

    def drain_in(b):
        pltpu.make_async_copy(x_hbm.at[pl.ds(0, SROWS)], xbuf.at[b],
                              sem_in.at[b]).wait()
        for h in range(2):
            pltpu.make_async_copy(batch_hbm.at[pl.ds(0, 80)],
                                  idxbuf.at[b].at[h], sem_in.at[b]).wait()

    for b in range(NBUF):
        issue_in(b, b)

    @pl.loop(0, ntrip, step=NBUF)
    def _group(g):
        for b in range(NBUF):
            k = g + b

            @pl.when(k < ntrip)
            def _visit():
                drain_in(b)

                for h in range(2):
                    @pl.loop(0, 80, step=UNROLL)
                    def _rows(r):
                        idxv = idxbuf[b, h, pl.ds(r, 16)]
                        plsc.addupdate_scatter(cntloc, [idxv], o16)
                        for i in range(UNROLL):
                            rowv = lax.broadcast(idxv[i], (16,))
                            for j in range(D // 16):
                                xv = xbuf[b, h * 80 + r + i, pl.ds(j * 16, 16)]
                                plsc.addupdate_scatter(accloc, [rowv, cols[j]], xv)

                @pl.when(k + NBUF < ntrip)
                def _refill():
                    issue_in(k + NBUF, b)

    pltpu.sync_copy(cntloc, cnts_out.at[wid])

    pltpu.sync_copy(accloc.at[pl.ds(0, 128)], acc.at[rowids.at[0]], add=True)
    pltpu.sync_copy(accloc.at[pl.ds(128, 128)], acc.at[rowids.at[1]], add=True)

    plsc.subcore_barrier()

    pltpu.sync_copy(acc.at[pl.ds(sid * 16, 16)], stage)
    pltpu.sync_copy(stage, sums_out.at[cid, pl.ds(sid * 16, 16)])


_seg_call = functools.partial(
    pl.kernel,
    out_type=[
        jax.ShapeDtypeStruct((NC, NSEG, D), jnp.float32),
        jax.ShapeDtypeStruct((NW, NSEG), jnp.float32),
    ],
    mesh=plsc.VectorSubcoreMesh(core_axis_name="c", subcore_axis_name="s",
                                num_cores=NC, num_subcores=NS),
    scratch_types=[
        pltpu.VMEM((NBUF, SROWS, D), jnp.float32),
        pltpu.VMEM((NBUF, 2, 80), jnp.int32),
        pltpu.VMEM((NSEG, D), jnp.float32),
        pltpu.VMEM((NSEG,), jnp.float32),
        pltpu.VMEM((2, 128), jnp.int32),
        pltpu.VMEM((16, D), jnp.float32),
        pltpu.VMEM_SHARED((NSEG, D), jnp.float32),
        pltpu.SemaphoreType.DMA((NBUF,)),
    ],
    compiler_params=pltpu.CompilerParams(needs_layout_passes=False),
)(_seg_body)


def _mlp_body(sums_ref, cnts_ref, u_ref, W1_ref, b1_ref, gamma_ref,
              beta_ref, W2_ref, b2_ref, out_ref):
    sums = sums_ref[0] + sums_ref[1]
    cnt = jnp.sum(cnts_ref[...], axis=0)[:, None]
    mean = sums / jnp.maximum(cnt, 1.0)
    g_in = u_ref.shape[1]
    W1u = W1_ref[0:g_in, :]
    W1m = W1_ref[g_in:, :]
    h = (jnp.dot(u_ref[...], W1u, preferred_element_type=jnp.float32)
         + jnp.dot(mean, W1m, preferred_element_type=jnp.float32)
         + b1_ref[...])
    mu = jnp.mean(h, axis=-1, keepdims=True)
    var = jnp.mean((h - mu) ** 2, axis=-1, keepdims=True)
    h = (h - mu) * lax.rsqrt(var + 1e-5) * gamma_ref[...] + beta_ref[...]
    h = jnp.maximum(h, 0.0)
    out_ref[...] = (jnp.dot(h, W2_ref[...], preferred_element_type=jnp.float32)
                    + b2_ref[...])


def kernel(x, edge_index, edge_attr, u, batch, W1, b1, gamma, beta, W2, b2):
    del edge_index, edge_attr
    sums, cnts = _seg_call(x, batch)
    out = pl.pallas_call(
        _mlp_body,
        out_shape=jax.ShapeDtypeStruct((u.shape[0], W2.shape[1]), jnp.float32),
    )(sums, cnts, u, W1, b1, gamma, beta, W2, b2)
    return out


# speedup vs baseline: 2.4222x; 2.4222x over previous
import functools

import jax
import jax.numpy as jnp
from jax import lax
from jax.experimental import pallas as pl
from jax.experimental.pallas import tpu as pltpu
from jax.experimental.pallas import tpu_sc as plsc

N = 100000
D = 128
NSEG = 256
CHUNK = 80
SUPER = 2
SROWS = SUPER * CHUNK
NSUPER = N // SROWS
NBUF = 4
NC = 2
NS = 16
NW = NC * NS


def _seg_body(x_hbm, batch_hbm, sums_out, cnts_out,
              xbuf, idxbuf, cntloc, stage, acc, sem_in, sem_sc):
    cid = lax.axis_index("c")
    sid = lax.axis_index("s")
    wid = sid * NC + cid

    z16 = jnp.zeros((16,), jnp.float32)
    for r in range(16):
        for j in range(D // 16):
            stage[r, pl.ds(j * 16, 16)] = z16
    for j in range(NSEG // 16):
        cntloc[pl.ds(j * 16, 16)] = z16
    pltpu.sync_copy(stage, acc.at[pl.ds(sid * 16, 16)])

    plsc.subcore_barrier()

    base = NSUPER // NW
    rem = NSUPER - base * NW
    ntrip = base + jnp.where(wid < rem, 1, 0)
    o16 = jnp.ones((16,), jnp.float32)

    def issue_in(k, b):
        s = wid + k * NW
        pltpu.async_copy(x_hbm.at[pl.ds(s * SROWS, SROWS)], xbuf.at[b],
                         sem_in.at[b])
        pltpu.async_copy(batch_hbm.at[pl.ds(s * SUPER, SUPER)], idxbuf.at[b],
                         sem_in.at[b])

    for b in range(NBUF):
        issue_in(b, b)

    @pl.loop(0, ntrip, step=NBUF)
    def _group(g):
        for b in range(NBUF):
            k = g + b

            @pl.when(k < ntrip)
            def _visit():
                pltpu.make_async_copy(x_hbm.at[pl.ds(0, SROWS)], xbuf.at[b],
                                      sem_in.at[b]).wait()
                pltpu.make_async_copy(batch_hbm.at[pl.ds(0, SUPER)],
                                      idxbuf.at[b], sem_in.at[b]).wait()
                for j in range(SUPER):
                    for l in range(CHUNK // 16):
                        idxv = idxbuf[b, j, pl.ds(l * 16, 16)]
                        plsc.addupdate_scatter(cntloc, [idxv], o16)
                descs = []
                for j in range(SUPER):
                    descs.append(pltpu.async_copy(
                        xbuf.at[b].at[pl.ds(j * CHUNK, CHUNK)],
                        acc.at[idxbuf.at[b].at[j]],
                        sem_sc.at[b], add=True))
                for d in descs:
                    d.wait()
                @pl.when(k + NBUF < ntrip)
                def _next():
                    issue_in(k + NBUF, b)

    pltpu.sync_copy(cntloc, cnts_out.at[wid])

    plsc.subcore_barrier()

    pltpu.sync_copy(acc.at[pl.ds(sid * 16, 16)], stage)
    pltpu.sync_copy(stage, sums_out.at[cid, pl.ds(sid * 16, 16)])


_seg_call = functools.partial(
    pl.kernel,
    out_type=[
        jax.ShapeDtypeStruct((NC, NSEG, D), jnp.float32),
        jax.ShapeDtypeStruct((NW, NSEG), jnp.float32),
    ],
    mesh=plsc.VectorSubcoreMesh(core_axis_name="c", subcore_axis_name="s",
                                num_cores=NC, num_subcores=NS),
    scratch_types=[
        pltpu.VMEM((NBUF, SROWS, D), jnp.float32),
        pltpu.VMEM((NBUF, SUPER, CHUNK), jnp.int32),
        pltpu.VMEM((NSEG,), jnp.float32),
        pltpu.VMEM((16, D), jnp.float32),
        pltpu.VMEM_SHARED((NSEG, D), jnp.float32),
        pltpu.SemaphoreType.DMA((NBUF,)),
        pltpu.SemaphoreType.DMA((NBUF,)),
    ],
    compiler_params=pltpu.CompilerParams(needs_layout_passes=False),
)(_seg_body)


def _mlp_body(sums_ref, cnts_ref, u_ref, W1_ref, b1_ref, gamma_ref,
              beta_ref, W2_ref, b2_ref, out_ref):
    sums = sums_ref[0] + sums_ref[1]
    cnt = jnp.sum(cnts_ref[...], axis=0)[:, None]
    mean = sums / jnp.maximum(cnt, 1.0)
    g_in = u_ref.shape[1]
    W1u = W1_ref[0:g_in, :]
    W1m = W1_ref[g_in:, :]
    h = (jnp.dot(u_ref[...], W1u, preferred_element_type=jnp.float32)
         + jnp.dot(mean, W1m, preferred_element_type=jnp.float32)
         + b1_ref[...])
    mu = jnp.mean(h, axis=-1, keepdims=True)
    var = jnp.mean((h - mu) ** 2, axis=-1, keepdims=True)
    h = (h - mu) * lax.rsqrt(var + 1e-5) * gamma_ref[...] + beta_ref[...]
    h = jnp.maximum(h, 0.0)
    out_ref[...] = (jnp.dot(h, W2_ref[...], preferred_element_type=jnp.float32)
                    + b2_ref[...])


def kernel(x, edge_index, edge_attr, u, batch, W1, b1, gamma, beta, W2, b2):
    del edge_index, edge_attr
    batch2 = batch.reshape(NSUPER * SUPER, CHUNK)
    sums, cnts = _seg_call(x, batch2)
    out = pl.pallas_call(
        _mlp_body,
        out_shape=jax.ShapeDtypeStruct((u.shape[0], W2.shape[1]), jnp.float32),
    )(sums, cnts, u, W1, b1, gamma, beta, W2, b2)
    return out


# speedup vs baseline: 2.4308x; 1.0036x over previous
import functools

import jax
import jax.numpy as jnp
from jax import lax
from jax.experimental import pallas as pl
from jax.experimental.pallas import tpu as pltpu
from jax.experimental.pallas import tpu_sc as plsc

N = 100000
D = 128
NSEG = 256
CHUNK = 80
SUPER = 2
SROWS = SUPER * CHUNK
NSUPER = N // SROWS
NBUF = 4
NC = 2
NS = 16
NW = NC * NS


def _seg_body(x_hbm, batch_hbm, sums_out, cnts_out,
              xbuf, idxbuf, cntloc, stage, acc, sem_in, sem_sc):
    cid = lax.axis_index("c")
    sid = lax.axis_index("s")
    wid = sid * NC + cid

    z16 = jnp.zeros((16,), jnp.float32)
    for r in range(16):
        for j in range(D // 16):
            stage[r, pl.ds(j * 16, 16)] = z16
    for j in range(NSEG // 16):
        cntloc[pl.ds(j * 16, 16)] = z16
    pltpu.sync_copy(stage, acc.at[pl.ds(sid * 16, 16)])

    plsc.subcore_barrier()

    base = NSUPER // NW
    rem = NSUPER - base * NW
    ntrip = base + jnp.where(wid < rem, 1, 0)
    o16 = jnp.ones((16,), jnp.float32)

    def issue_in(k, b):
        s = wid + k * NW
        pltpu.async_copy(x_hbm.at[pl.ds(s * SROWS, SROWS)], xbuf.at[b],
                         sem_in.at[b])
        for h in range(SUPER):
            pltpu.async_copy(batch_hbm.at[pl.ds(s * SROWS + h * CHUNK, CHUNK)],
                             idxbuf.at[b].at[h], sem_in.at[b])

    for b in range(NBUF):
        issue_in(b, b)

    @pl.loop(0, ntrip, step=NBUF)
    def _group(g):
        for b in range(NBUF):
            k = g + b

            @pl.when(k < ntrip)
            def _visit():
                pltpu.make_async_copy(x_hbm.at[pl.ds(0, SROWS)], xbuf.at[b],
                                      sem_in.at[b]).wait()
                for h in range(SUPER):
                    pltpu.make_async_copy(batch_hbm.at[pl.ds(0, CHUNK)],
                                          idxbuf.at[b].at[h],
                                          sem_in.at[b]).wait()
                for j in range(SUPER):
                    for l in range(CHUNK // 16):
                        idxv = idxbuf[b, j, pl.ds(l * 16, 16)]
                        plsc.addupdate_scatter(cntloc, [idxv], o16)
                descs = []
                for j in range(SUPER):
                    descs.append(pltpu.async_copy(
                        xbuf.at[b].at[pl.ds(j * CHUNK, CHUNK)],
                        acc.at[idxbuf.at[b].at[j]],
                        sem_sc.at[b], add=True))
                for d in descs:
                    d.wait()
                @pl.when(k + NBUF < ntrip)
                def _next():
                    issue_in(k + NBUF, b)

    pltpu.sync_copy(cntloc, cnts_out.at[wid])

    plsc.subcore_barrier()

    pltpu.sync_copy(acc.at[pl.ds(sid * 16, 16)], stage)
    pltpu.sync_copy(stage, sums_out.at[cid, pl.ds(sid * 16, 16)])


_seg_call = functools.partial(
    pl.kernel,
    out_type=[
        jax.ShapeDtypeStruct((NC, NSEG, D), jnp.float32),
        jax.ShapeDtypeStruct((NW, NSEG), jnp.float32),
    ],
    mesh=plsc.VectorSubcoreMesh(core_axis_name="c", subcore_axis_name="s",
                                num_cores=NC, num_subcores=NS),
    scratch_types=[
        pltpu.VMEM((NBUF, SROWS, D), jnp.float32),
        pltpu.VMEM((NBUF, SUPER, CHUNK), jnp.int32),
        pltpu.VMEM((NSEG,), jnp.float32),
        pltpu.VMEM((16, D), jnp.float32),
        pltpu.VMEM_SHARED((NSEG, D), jnp.float32),
        pltpu.SemaphoreType.DMA((NBUF,)),
        pltpu.SemaphoreType.DMA((NBUF,)),
    ],
    compiler_params=pltpu.CompilerParams(needs_layout_passes=False),
)(_seg_body)


def _mlp_body(sums_ref, cnts_ref, u_ref, W1_ref, b1_ref, gamma_ref,
              beta_ref, W2_ref, b2_ref, out_ref):
    sums = sums_ref[0] + sums_ref[1]
    cnt = jnp.sum(cnts_ref[...], axis=0)[:, None]
    mean = sums / jnp.maximum(cnt, 1.0)
    g_in = u_ref.shape[1]
    W1u = W1_ref[0:g_in, :]
    W1m = W1_ref[g_in:, :]
    h = (jnp.dot(u_ref[...], W1u, preferred_element_type=jnp.float32)
         + jnp.dot(mean, W1m, preferred_element_type=jnp.float32)
         + b1_ref[...])
    mu = jnp.mean(h, axis=-1, keepdims=True)
    var = jnp.mean((h - mu) ** 2, axis=-1, keepdims=True)
    h = (h - mu) * lax.rsqrt(var + 1e-5) * gamma_ref[...] + beta_ref[...]
    h = jnp.maximum(h, 0.0)
    out_ref[...] = (jnp.dot(h, W2_ref[...], preferred_element_type=jnp.float32)
                    + b2_ref[...])


def kernel(x, edge_index, edge_attr, u, batch, W1, b1, gamma, beta, W2, b2):
    del edge_index, edge_attr
    sums, cnts = _seg_call(x, batch)
    out = pl.pallas_call(
        _mlp_body,
        out_shape=jax.ShapeDtypeStruct((u.shape[0], W2.shape[1]), jnp.float32),
    )(sums, cnts, u, W1, b1, gamma, beta, W2, b2)
    return out
